# baseline (speedup 1.0000x reference)
import jax
import jax.numpy as jnp
from jax import lax
from jax.experimental import pallas as pl
from jax.experimental.pallas import tpu as pltpu
from jax.experimental.pallas import tpu_sc as plsc

N = 10000
E = 320000
D = 128
NC = 2
NS = 16
NW = NC * NS
CHUNK = 64
NG = 3
NSB = 2
NPH = 4
PH = 40
NCHUNK = NPH * PH
EPT = NCHUNK * CHUNK
EPP = PH * CHUNK
EPAD = NW * EPT
NP = 10240
RPT = NP // NS


def _sc_body(xb_hbm, src_hbm, dst_hbm, w_hbm, agg_hbm,
             src_v, dst_v, w_v, gb0, gb1, gb2, sb0, sb1, agg_spmem,
             g0, g1, g2, s0, s1, zsem):
    gbufs = (gb0, gb1, gb2)
    sbufs = (sb0, sb1)
    gsems = (g0, g1, g2)
    ssems = (s0, s1)
    cid = lax.axis_index("c")
    sid = lax.axis_index("s")
    wid = cid * NS + sid

    zero16 = jnp.zeros((16,), jnp.float32)

    def zrow(r, _):
        for j in range(D // 16):
            sb0[r, pl.ds(j * 16, 16)] = zero16
        return 0

    lax.fori_loop(0, CHUNK, zrow, 0)
    for q in range(RPT // CHUNK):
        pltpu.async_copy(sb0,
                         agg_spmem.at[pl.ds(sid * RPT + q * CHUNK, CHUNK)],
                         zsem)
    for q in range(RPT // CHUNK):
        pltpu.make_async_copy(
            sb0, agg_spmem.at[pl.ds(sid * RPT + q * CHUNK, CHUNK)],
            zsem).wait()
    plsc.subcore_barrier()

    def start_gather(c, kg):
        pltpu.async_copy(xb_hbm.at[src_v.at[pl.ds(c * CHUNK, CHUNK)]],
                         gbufs[kg], gsems[kg])

    def wait_gather(kg):
        pltpu.make_async_copy(xb_hbm.at[src_v.at[pl.ds(0, CHUNK)]],
                              gbufs[kg], gsems[kg]).wait()

    def start_scatter(c, ks):
        pltpu.async_copy(sbufs[ks], agg_spmem.at[dst_v.at[c]], ssems[ks],
                         add=True)

    def wait_scatter(ks):
        pltpu.make_async_copy(sbufs[ks], agg_spmem.at[dst_v.at[0]],
                              ssems[ks]).wait()

    mask_hi = jnp.full((16,), -65536, jnp.int32)

    def scale(c, gbuf, sbuf):
        base = c * CHUNK

        def gbody(g, _):
            w16 = w_v[pl.ds(pl.multiple_of(base + g * 16, 16), 16)]
            for lane in range(16):
                we = jnp.full((16,), w16[lane], jnp.float32)
                row = g * 16 + lane
                for j in range(D // 32):
                    w32 = gbuf[row, pl.ds(j * 16, 16)]
                    lo = lax.bitcast_convert_type(
                        lax.shift_left(w32, 16), jnp.float32)
                    hi = lax.bitcast_convert_type(w32 & mask_hi, jnp.float32)
                    sbuf[row, pl.ds(j * 32, 16)] = lo * we
                    sbuf[row, pl.ds(j * 32 + 16, 16)] = hi * we
            return 0

        lax.fori_loop(0, CHUNK // 16, gbody, 0)

    def phase(p, _):
        pltpu.sync_copy(src_hbm.at[pl.ds(wid * EPT + p * EPP, EPP)], src_v)
        pltpu.sync_copy(dst_hbm.at[wid * NPH + p], dst_v)
        pltpu.sync_copy(w_hbm.at[pl.ds(wid * EPT + p * EPP, EPP)], w_v)

        start_gather(0, 0)
        start_gather(1, 1)
        for c in range(2):
            wait_gather(c)
            start_gather(c + 2, (c + 2) % NG)
            scale(c, gbufs[c], sbufs[c])
            start_scatter(c, c)

        def step(ii, _):
            for j6 in range(6):
                c = 2 + ii * 6 + j6
                kg = (2 + j6) % NG
                ks = j6 % NSB
                wait_gather(kg)
                start_gather(c + 2, (kg + 2) % NG)
                wait_scatter(ks)
                scale(c, gbufs[kg], sbufs[ks])
                start_scatter(c, ks)
            return 0

        lax.fori_loop(0, (PH - 4) // 6, step, 0)

        for c in range(PH - 2, PH):
            kg = c % NG
            ks = c % NSB
            wait_gather(kg)
            wait_scatter(ks)
            scale(c, gbufs[kg], sbufs[ks])
            start_scatter(c, ks)

        for ks in range(NSB):
            wait_scatter(ks)
        return 0

    lax.fori_loop(0, NPH, phase, 0)

    plsc.subcore_barrier()
    pltpu.sync_copy(agg_spmem.at[pl.ds(sid * RPT, RPT)],
                    agg_hbm.at[cid, pl.ds(sid * RPT, RPT)])


_sc_call = pl.kernel(
    _sc_body,
    out_type=jax.ShapeDtypeStruct((NC, NP, D), jnp.float32),
    mesh=plsc.VectorSubcoreMesh(core_axis_name="c", subcore_axis_name="s",
                                num_cores=NC, num_subcores=NS),
    compiler_params=pltpu.CompilerParams(use_tc_tiling_on_sc=False),
    scratch_types=[
        pltpu.VMEM((EPP,), jnp.int32),
        pltpu.VMEM((PH, CHUNK), jnp.int32),
        pltpu.VMEM((EPP,), jnp.float32),
        pltpu.VMEM((CHUNK, D // 2), jnp.int32),
        pltpu.VMEM((CHUNK, D // 2), jnp.int32),
        pltpu.VMEM((CHUNK, D // 2), jnp.int32),
        pltpu.VMEM((CHUNK, D), jnp.float32),
        pltpu.VMEM((CHUNK, D), jnp.float32),
        pltpu.VMEM_SHARED((NP, D), jnp.float32),
        pltpu.SemaphoreType.DMA,
        pltpu.SemaphoreType.DMA,
        pltpu.SemaphoreType.DMA,
        pltpu.SemaphoreType.DMA,
        pltpu.SemaphoreType.DMA,
        pltpu.SemaphoreType.DMA,
    ],
)


def _mlp_body(x_ref, agg_ref, w1_ref, w2_ref, b_ref, o_ref):
    acc = jnp.dot(x_ref[...], w1_ref[...], preferred_element_type=jnp.float32)
    acc = acc + jnp.dot(agg_ref[0] + agg_ref[1], w2_ref[...],
                        preferred_element_type=jnp.float32)
    o_ref[...] = jnp.maximum(acc + b_ref[...], 0.0)


def kernel(x, edge_index, edge_weight, W, b):
    src = edge_index[0].astype(jnp.int32)
    dst = edge_index[1].astype(jnp.int32)
    w = edge_weight.astype(jnp.float32)

    xb = jax.lax.bitcast_convert_type(
        x.astype(jnp.bfloat16)
        .reshape(N, D // 32, 2, 16)
        .transpose(0, 1, 3, 2)
        .reshape(N, D // 2, 2),
        jnp.int32)

    pad = EPAD - E
    fill = (jnp.arange(pad, dtype=jnp.int32) * 97) % N
    src_p = jnp.concatenate([src, fill])
    dst_p = jnp.concatenate([dst, fill]).reshape(NW * NPH, PH, CHUNK)
    w_p = jnp.concatenate([w, jnp.zeros((pad,), jnp.float32)])

    agg = _sc_call(xb, src_p, dst_p, w_p)

    w1 = W[:D]
    w2 = W[D:]
    b2 = b.reshape(1, D)
    rows_blk = 1000
    out = pl.pallas_call(
        _mlp_body,
        grid=(N // rows_blk,),
        in_specs=[
            pl.BlockSpec((rows_blk, D), lambda i: (i, 0)),
            pl.BlockSpec((NC, rows_blk, D), lambda i: (0, i, 0)),
            pl.BlockSpec((D, D), lambda i: (0, 0)),
            pl.BlockSpec((D, D), lambda i: (0, 0)),
            pl.BlockSpec((1, D), lambda i: (0, 0)),
        ],
        out_specs=pl.BlockSpec((rows_blk, D), lambda i: (i, 0)),
        out_shape=jax.ShapeDtypeStruct((N, D), jnp.float32),
    )(x, agg, w1, w2, b2)
    return out


# speedup vs baseline: 1.8096x; 1.8096x over previous
import jax
import jax.numpy as jnp
from jax import lax
from jax.experimental import pallas as pl
from jax.experimental.pallas import tpu as pltpu
from jax.experimental.pallas import tpu_sc as plsc

N = 10000
E = 320000
D = 128
NC = 2
NS = 16
NW = NC * NS
CHUNK = 64
NG = 3
NSB = 2
NPH = 4
PH = 40
NCHUNK = NPH * PH
EPT = NCHUNK * CHUNK
EPP = PH * CHUNK
EPAD = NW * EPT
NP = 10240
RPT = NP // NS


def _sc_body(xb_hbm, src_hbm, dst_hbm, w_hbm, agg_hbm,
             src_v, dst_v, w_v, gb0, gb1, gb2, sb0, sb1, agg_spmem,
             g0, g1, g2, s0, s1, zsem):
    gbufs = (gb0, gb1, gb2)
    sbufs = (sb0, sb1)
    gsems = (g0, g1, g2)
    ssems = (s0, s1)
    cid = lax.axis_index("c")
    sid = lax.axis_index("s")
    wid = cid * NS + sid

    zero16 = jnp.zeros((16,), jnp.float32)

    def zrow(r, _):
        for j in range(D // 16):
            sb0[r, pl.ds(j * 16, 16)] = zero16
        return 0

    lax.fori_loop(0, CHUNK, zrow, 0)
    for q in range(RPT // CHUNK):
        pltpu.async_copy(sb0,
                         agg_spmem.at[pl.ds(sid * RPT + q * CHUNK, CHUNK)],
                         zsem)
    for q in range(RPT // CHUNK):
        pltpu.make_async_copy(
            sb0, agg_spmem.at[pl.ds(sid * RPT + q * CHUNK, CHUNK)],
            zsem).wait()
    plsc.subcore_barrier()

    def start_gather(c, kg):
        pltpu.async_copy(xb_hbm.at[src_v.at[pl.ds(c * CHUNK, CHUNK)]],
                         gbufs[kg], gsems[kg])

    def wait_gather(kg):
        pltpu.make_async_copy(xb_hbm.at[src_v.at[pl.ds(0, CHUNK)]],
                              gbufs[kg], gsems[kg]).wait()

    def start_scatter(c, ks):
        pltpu.async_copy(sbufs[ks], agg_spmem.at[dst_v.at[c]], ssems[ks],
                         add=True)

    def wait_scatter(ks):
        pltpu.make_async_copy(sbufs[ks], agg_spmem.at[dst_v.at[0]],
                              ssems[ks]).wait()

    mask_hi = jnp.full((16,), -65536, jnp.int32)

    def scale(c, gbuf, sbuf):
        base = c * CHUNK

        def gbody(g, _):
            w16 = w_v[pl.ds(pl.multiple_of(base + g * 16, 16), 16)]
            for lane in range(16):
                we = jnp.full((16,), w16[lane], jnp.float32)
                row = g * 16 + lane
                ws = [gbuf[row, pl.ds(j * 16, 16)] for j in range(D // 32)]
                los = [lax.bitcast_convert_type(lax.shift_left(w32, 16),
                                                jnp.float32) * we
                       for w32 in ws]
                his = [lax.bitcast_convert_type(w32 & mask_hi,
                                                jnp.float32) * we
                       for w32 in ws]
                for j in range(D // 32):
                    sbuf[row, pl.ds(j * 32, 16)] = los[j]
                    sbuf[row, pl.ds(j * 32 + 16, 16)] = his[j]
            return 0

        lax.fori_loop(0, CHUNK // 16, gbody, 0)

    def phase(p, _):
        pltpu.sync_copy(src_hbm.at[pl.ds(wid * EPT + p * EPP, EPP)], src_v)
        pltpu.sync_copy(dst_hbm.at[wid * NPH + p], dst_v)
        pltpu.sync_copy(w_hbm.at[pl.ds(wid * EPT + p * EPP, EPP)], w_v)

        start_gather(0, 0)
        start_gather(1, 1)
        for c in range(2):
            wait_gather(c)
            start_gather(c + 2, (c + 2) % NG)
            scale(c, gbufs[c], sbufs[c])
            start_scatter(c, c)

        def step(ii, _):
            for j6 in range(6):
                c = 2 + ii * 6 + j6
                kg = (2 + j6) % NG
                ks = j6 % NSB
                wait_gather(kg)
                start_gather(c + 2, (kg + 2) % NG)
                wait_scatter(ks)
                scale(c, gbufs[kg], sbufs[ks])
                start_scatter(c, ks)
            return 0

        lax.fori_loop(0, (PH - 4) // 6, step, 0)

        for c in range(PH - 2, PH):
            kg = c % NG
            ks = c % NSB
            wait_gather(kg)
            wait_scatter(ks)
            scale(c, gbufs[kg], sbufs[ks])
            start_scatter(c, ks)

        for ks in range(NSB):
            wait_scatter(ks)
        return 0

    lax.fori_loop(0, NPH, phase, 0)

    plsc.subcore_barrier()
    pltpu.sync_copy(agg_spmem.at[pl.ds(sid * RPT, RPT)],
                    agg_hbm.at[cid, pl.ds(sid * RPT, RPT)])


_sc_call = pl.kernel(
    _sc_body,
    out_type=jax.ShapeDtypeStruct((NC, NP, D), jnp.float32),
    mesh=plsc.VectorSubcoreMesh(core_axis_name="c", subcore_axis_name="s",
                                num_cores=NC, num_subcores=NS),
    compiler_params=pltpu.CompilerParams(use_tc_tiling_on_sc=False),
    scratch_types=[
        pltpu.VMEM((EPP,), jnp.int32),
        pltpu.VMEM((PH, CHUNK), jnp.int32),
        pltpu.VMEM((EPP,), jnp.float32),
        pltpu.VMEM((CHUNK, D // 2), jnp.int32),
        pltpu.VMEM((CHUNK, D // 2), jnp.int32),
        pltpu.VMEM((CHUNK, D // 2), jnp.int32),
        pltpu.VMEM((CHUNK, D), jnp.float32),
        pltpu.VMEM((CHUNK, D), jnp.float32),
        pltpu.VMEM_SHARED((NP, D), jnp.float32),
        pltpu.SemaphoreType.DMA,
        pltpu.SemaphoreType.DMA,
        pltpu.SemaphoreType.DMA,
        pltpu.SemaphoreType.DMA,
        pltpu.SemaphoreType.DMA,
        pltpu.SemaphoreType.DMA,
    ],
)


def _mlp_body(x_ref, agg_ref, w1_ref, w2_ref, b_ref, o_ref):
    acc = jnp.dot(x_ref[...], w1_ref[...], preferred_element_type=jnp.float32)
    acc = acc + jnp.dot(agg_ref[0] + agg_ref[1], w2_ref[...],
                        preferred_element_type=jnp.float32)
    o_ref[...] = jnp.maximum(acc + b_ref[...], 0.0)


def kernel(x, edge_index, edge_weight, W, b):
    src = edge_index[0].astype(jnp.int32)
    dst = edge_index[1].astype(jnp.int32)
    w = edge_weight.astype(jnp.float32)

    xb = jax.lax.bitcast_convert_type(
        x.astype(jnp.bfloat16)
        .reshape(N, D // 32, 2, 16)
        .transpose(0, 1, 3, 2)
        .reshape(N, D // 2, 2),
        jnp.int32)

    pad = EPAD - E
    fill = (jnp.arange(pad, dtype=jnp.int32) * 97) % N
    src_p = jnp.concatenate([src, fill])
    dst_p = jnp.concatenate([dst, fill]).reshape(NW * NPH, PH, CHUNK)
    w_p = jnp.concatenate([w, jnp.zeros((pad,), jnp.float32)])

    agg = _sc_call(xb, src_p, dst_p, w_p)

    w1 = W[:D]
    w2 = W[D:]
    b2 = b.reshape(1, D)
    rows_blk = 1000
    out = pl.pallas_call(
        _mlp_body,
        grid=(N // rows_blk,),
        in_specs=[
            pl.BlockSpec((rows_blk, D), lambda i: (i, 0)),
            pl.BlockSpec((NC, rows_blk, D), lambda i: (0, i, 0)),
            pl.BlockSpec((D, D), lambda i: (0, 0)),
            pl.BlockSpec((D, D), lambda i: (0, 0)),
            pl.BlockSpec((1, D), lambda i: (0, 0)),
        ],
        out_specs=pl.BlockSpec((rows_blk, D), lambda i: (i, 0)),
        out_shape=jax.ShapeDtypeStruct((N, D), jnp.float32),
    )(x, agg, w1, w2, b2)
    return out


# speedup vs baseline: 1.9205x; 1.0613x over previous
import jax
import jax.numpy as jnp
from jax import lax
from jax.experimental import pallas as pl
from jax.experimental.pallas import tpu as pltpu
from jax.experimental.pallas import tpu_sc as plsc

N = 10000
E = 320000
D = 128
NC = 2
NS = 16
NW = NC * NS
CHUNK = 64
NG = 3
NSB = 2
NPH = 4
PH = 40
NCHUNK = NPH * PH
EPT = NCHUNK * CHUNK
EPP = PH * CHUNK
EPAD = NW * EPT
NP = 10240
RPT = NP // NS


def _sc_body(xb_hbm, ei_hbm, w_hbm, agg_hbm,
             src_v, dst_v, w_v, gb0, gb1, gb2, sb0, sb1, agg_spmem,
             g0, g1, g2, s0, s1, zsem):
    gbufs = (gb0, gb1, gb2)
    sbufs = (sb0, sb1)
    gsems = (g0, g1, g2)
    ssems = (s0, s1)
    cid = lax.axis_index("c")
    sid = lax.axis_index("s")
    wid = cid * NS + sid

    zero16 = jnp.zeros((16,), jnp.float32)

    def zrow(r, _):
        for j in range(D // 16):
            sb0[r, pl.ds(j * 16, 16)] = zero16
        return 0

    lax.fori_loop(0, CHUNK, zrow, 0)
    for q in range(RPT // CHUNK):
        pltpu.async_copy(sb0,
                         agg_spmem.at[pl.ds(sid * RPT + q * CHUNK, CHUNK)],
                         zsem)
    for q in range(RPT // CHUNK):
        pltpu.make_async_copy(
            sb0, agg_spmem.at[pl.ds(sid * RPT + q * CHUNK, CHUNK)],
            zsem).wait()
    plsc.subcore_barrier()

    def start_gather(c, kg):
        pltpu.async_copy(xb_hbm.at[src_v.at[pl.ds(c * CHUNK, CHUNK)]],
                         gbufs[kg], gsems[kg])

    def wait_gather(kg):
        pltpu.make_async_copy(xb_hbm.at[src_v.at[pl.ds(0, CHUNK)]],
                              gbufs[kg], gsems[kg]).wait()

    def start_scatter(c, ks):
        pltpu.async_copy(sbufs[ks],
                         agg_spmem.at[dst_v.at[pl.ds(c * CHUNK, CHUNK)]],
                         ssems[ks], add=True)

    def wait_scatter(ks):
        pltpu.make_async_copy(sbufs[ks],
                              agg_spmem.at[dst_v.at[pl.ds(0, CHUNK)]],
                              ssems[ks]).wait()

    mask_hi = jnp.full((16,), -65536, jnp.int32)

    def scale(c, gbuf, sbuf):
        base = c * CHUNK

        def gbody(g, _):
            w16 = w_v[pl.ds(pl.multiple_of(base + g * 16, 16), 16)]
            wes = [jnp.full((16,), w16[lane], jnp.float32)
                   for lane in range(16)]
            for lane in range(16):
                we = wes[lane]
                row = g * 16 + lane
                ws = [gbuf[row, pl.ds(j * 16, 16)] for j in range(D // 32)]
                los = [lax.bitcast_convert_type(lax.shift_left(w32, 16),
                                                jnp.float32) * we
                       for w32 in ws]
                his = [lax.bitcast_convert_type(w32 & mask_hi,
                                                jnp.float32) * we
                       for w32 in ws]
                for j in range(D // 32):
                    sbuf[row, pl.ds(j * 32, 16)] = los[j]
                    sbuf[row, pl.ds(j * 32 + 16, 16)] = his[j]
            return 0

        lax.fori_loop(0, CHUNK // 16, gbody, 0)

    def phase(p, _):
        pltpu.sync_copy(ei_hbm.at[0, pl.ds(wid * EPT + p * EPP, EPP)], src_v)
        pltpu.sync_copy(ei_hbm.at[1, pl.ds(wid * EPT + p * EPP, EPP)], dst_v)
        pltpu.sync_copy(w_hbm.at[pl.ds(wid * EPT + p * EPP, EPP)], w_v)

        start_gather(0, 0)
        start_gather(1, 1)
        for c in range(2):
            wait_gather(c)
            start_gather(c + 2, (c + 2) % NG)
            scale(c, gbufs[c], sbufs[c])
            start_scatter(c, c)

        def step(ii, _):
            for j6 in range(6):
                c = 2 + ii * 6 + j6
                kg = (2 + j6) % NG
                ks = j6 % NSB
                wait_gather(kg)
                start_gather(c + 2, (kg + 2) % NG)
                wait_scatter(ks)
                scale(c, gbufs[kg], sbufs[ks])
                start_scatter(c, ks)
            return 0

        lax.fori_loop(0, (PH - 4) // 6, step, 0)

        for c in range(PH - 2, PH):
            kg = c % NG
            ks = c % NSB
            wait_gather(kg)
            wait_scatter(ks)
            scale(c, gbufs[kg], sbufs[ks])
            start_scatter(c, ks)

        for ks in range(NSB):
            wait_scatter(ks)
        return 0

    lax.fori_loop(0, NPH, phase, 0)

    plsc.subcore_barrier()
    pltpu.sync_copy(agg_spmem.at[pl.ds(sid * RPT, RPT)],
                    agg_hbm.at[cid, pl.ds(sid * RPT, RPT)])


_sc_call = pl.kernel(
    _sc_body,
    out_type=jax.ShapeDtypeStruct((NC, NP, D), jnp.float32),
    mesh=plsc.VectorSubcoreMesh(core_axis_name="c", subcore_axis_name="s",
                                num_cores=NC, num_subcores=NS),
    compiler_params=pltpu.CompilerParams(use_tc_tiling_on_sc=False),
    scratch_types=[
        pltpu.VMEM((EPP,), jnp.int32),
        pltpu.VMEM((EPP,), jnp.int32),
        pltpu.VMEM((EPP,), jnp.float32),
        pltpu.VMEM((CHUNK, D // 2), jnp.int32),
        pltpu.VMEM((CHUNK, D // 2), jnp.int32),
        pltpu.VMEM((CHUNK, D // 2), jnp.int32),
        pltpu.VMEM((CHUNK, D), jnp.float32),
        pltpu.VMEM((CHUNK, D), jnp.float32),
        pltpu.VMEM_SHARED((NP, D), jnp.float32),
        pltpu.SemaphoreType.DMA,
        pltpu.SemaphoreType.DMA,
        pltpu.SemaphoreType.DMA,
        pltpu.SemaphoreType.DMA,
        pltpu.SemaphoreType.DMA,
        pltpu.SemaphoreType.DMA,
    ],
)


def _mlp_body(x_ref, agg_ref, w1_ref, w2_ref, b_ref, o_ref):
    acc = jnp.dot(x_ref[...], w1_ref[...], preferred_element_type=jnp.float32)
    acc = acc + jnp.dot(agg_ref[0] + agg_ref[1], w2_ref[...],
                        preferred_element_type=jnp.float32)
    o_ref[...] = jnp.maximum(acc + b_ref[...], 0.0)


def kernel(x, edge_index, edge_weight, W, b):
    ei = edge_index.astype(jnp.int32)
    w = edge_weight.astype(jnp.float32)

    xb = jax.lax.bitcast_convert_type(
        x.astype(jnp.bfloat16)
        .reshape(N, D // 32, 2, 16)
        .transpose(0, 1, 3, 2)
        .reshape(N, D // 2, 2),
        jnp.int32)

    pad = EPAD - E
    fill = (jnp.arange(pad, dtype=jnp.int32) * 97) % N
    ei_p = jnp.concatenate([ei, jnp.stack([fill, fill])], axis=1)
    w_p = jnp.concatenate([w, jnp.zeros((pad,), jnp.float32)])

    agg = _sc_call(xb, ei_p, w_p)

    w1 = W[:D]
    w2 = W[D:]
    b2 = b.reshape(1, D)
    rows_blk = 1000
    out = pl.pallas_call(
        _mlp_body,
        grid=(N // rows_blk,),
        in_specs=[
            pl.BlockSpec((rows_blk, D), lambda i: (i, 0)),
            pl.BlockSpec((NC, rows_blk, D), lambda i: (0, i, 0)),
            pl.BlockSpec((D, D), lambda i: (0, 0)),
            pl.BlockSpec((D, D), lambda i: (0, 0)),
            pl.BlockSpec((1, D), lambda i: (0, 0)),
        ],
        out_specs=pl.BlockSpec((rows_blk, D), lambda i: (i, 0)),
        out_shape=jax.ShapeDtypeStruct((N, D), jnp.float32),
    )(x, agg, w1, w2, b2)
    return out


# speedup vs baseline: 2.1486x; 1.1188x over previous
import jax
import jax.numpy as jnp
from jax import lax
from jax.experimental import pallas as pl
from jax.experimental.pallas import tpu as pltpu
from jax.experimental.pallas import tpu_sc as plsc

N = 10000
E = 320000
D = 128
NC = 2
NS = 16
NW = NC * NS
CHUNK = 64
NG = 3
NSB = 2
NPH = 4
PH = 40
NCHUNK = NPH * PH
EPT = NCHUNK * CHUNK
EPP = PH * CHUNK
EPAD = NW * EPT
NP = 10240
RPT = NP // NS


def _sc_body(xb_hbm, ei_hbm, w_hbm, agg_hbm,
             src_v, dst_v, w_v, gb0, gb1, gb2, sb0, sb1, agg_spmem,
             g0, g1, g2, s0, s1, zsem):
    gbufs = (gb0, gb1, gb2)
    sbufs = (sb0, sb1)
    gsems = (g0, g1, g2)
    ssems = (s0, s1)
    cid = lax.axis_index("c")
    sid = lax.axis_index("s")
    wid = cid * NS + sid

    zero16 = jnp.zeros((16,), jnp.float32)

    def zrow(r, _):
        for j in range(D // 16):
            sb0[r, pl.ds(j * 16, 16)] = zero16
        return 0

    lax.fori_loop(0, CHUNK, zrow, 0)
    for q in range(RPT // CHUNK):
        pltpu.async_copy(sb0,
                         agg_spmem.at[pl.ds(sid * RPT + q * CHUNK, CHUNK)],
                         zsem)
    for q in range(RPT // CHUNK):
        pltpu.make_async_copy(
            sb0, agg_spmem.at[pl.ds(sid * RPT + q * CHUNK, CHUNK)],
            zsem).wait()
    plsc.subcore_barrier()

    def start_gather(c, kg):
        pltpu.async_copy(xb_hbm.at[src_v.at[pl.ds(c * CHUNK, CHUNK)]],
                         gbufs[kg], gsems[kg])

    def wait_gather(kg):
        pltpu.make_async_copy(xb_hbm.at[src_v.at[pl.ds(0, CHUNK)]],
                              gbufs[kg], gsems[kg]).wait()

    def start_scatter(c, ks):
        pltpu.async_copy(sbufs[ks],
                         agg_spmem.at[dst_v.at[pl.ds(c * CHUNK, CHUNK)]],
                         ssems[ks], add=True)

    def wait_scatter(ks):
        pltpu.make_async_copy(sbufs[ks],
                              agg_spmem.at[dst_v.at[pl.ds(0, CHUNK)]],
                              ssems[ks]).wait()

    mask_hi = jnp.full((16,), -65536, jnp.int32)

    def scale(c, gbuf, sbuf):
        base = c * CHUNK

        def gbody(g, _):
            w16 = w_v[pl.ds(pl.multiple_of(base + g * 16, 16), 16)]
            wes = [jnp.full((16,), w16[lane], jnp.float32)
                   for lane in range(16)]

            def loads(lane):
                row = g * 16 + lane
                return [gbuf[row, pl.ds(j * 16, 16)]
                        for j in range(D // 32)]

            def compute(ws, lane):
                we = wes[lane]
                los = [lax.bitcast_convert_type(lax.shift_left(w32, 16),
                                                jnp.float32) * we
                       for w32 in ws]
                his = [lax.bitcast_convert_type(w32, jnp.float32) * we
                       for w32 in ws]
                return los, his

            def stores(lane, los, his):
                row = g * 16 + lane
                for j in range(D // 32):
                    sbuf[row, pl.ds(j * 32, 16)] = los[j]
                    sbuf[row, pl.ds(j * 32 + 16, 16)] = his[j]

            ws = loads(0)
            for lane in range(16):
                nws = loads(lane + 1) if lane < 15 else None
                los, his = compute(ws, lane)
                stores(lane, los, his)
                ws = nws
            return 0

        lax.fori_loop(0, CHUNK // 16, gbody, 0)

    def phase(p, _):
        pltpu.sync_copy(ei_hbm.at[0, pl.ds(wid * EPT + p * EPP, EPP)], src_v)
        pltpu.sync_copy(ei_hbm.at[1, pl.ds(wid * EPT + p * EPP, EPP)], dst_v)
        pltpu.sync_copy(w_hbm.at[pl.ds(wid * EPT + p * EPP, EPP)], w_v)

        start_gather(0, 0)
        start_gather(1, 1)
        for c in range(2):
            wait_gather(c)
            start_gather(c + 2, (c + 2) % NG)
            scale(c, gbufs[c], sbufs[c])
            start_scatter(c, c)

        def step(ii, _):
            for j6 in range(6):
                c = 2 + ii * 6 + j6
                kg = (2 + j6) % NG
                ks = j6 % NSB
                wait_gather(kg)
                start_gather(c + 2, (kg + 2) % NG)
                wait_scatter(ks)
                scale(c, gbufs[kg], sbufs[ks])
                start_scatter(c, ks)
            return 0

        lax.fori_loop(0, (PH - 4) // 6, step, 0)

        for c in range(PH - 2, PH):
            kg = c % NG
            ks = c % NSB
            wait_gather(kg)
            wait_scatter(ks)
            scale(c, gbufs[kg], sbufs[ks])
            start_scatter(c, ks)

        for ks in range(NSB):
            wait_scatter(ks)
        return 0

    lax.fori_loop(0, NPH, phase, 0)

    plsc.subcore_barrier()
    pltpu.sync_copy(agg_spmem.at[pl.ds(sid * RPT, RPT)],
                    agg_hbm.at[cid, pl.ds(sid * RPT, RPT)])


_sc_call = pl.kernel(
    _sc_body,
    out_type=jax.ShapeDtypeStruct((NC, NP, D), jnp.float32),
    mesh=plsc.VectorSubcoreMesh(core_axis_name="c", subcore_axis_name="s",
                                num_cores=NC, num_subcores=NS),
    compiler_params=pltpu.CompilerParams(use_tc_tiling_on_sc=False),
    scratch_types=[
        pltpu.VMEM((EPP,), jnp.int32),
        pltpu.VMEM((EPP,), jnp.int32),
        pltpu.VMEM((EPP,), jnp.float32),
        pltpu.VMEM((CHUNK, D // 2), jnp.int32),
        pltpu.VMEM((CHUNK, D // 2), jnp.int32),
        pltpu.VMEM((CHUNK, D // 2), jnp.int32),
        pltpu.VMEM((CHUNK, D), jnp.float32),
        pltpu.VMEM((CHUNK, D), jnp.float32),
        pltpu.VMEM_SHARED((NP, D), jnp.float32),
        pltpu.SemaphoreType.DMA,
        pltpu.SemaphoreType.DMA,
        pltpu.SemaphoreType.DMA,
        pltpu.SemaphoreType.DMA,
        pltpu.SemaphoreType.DMA,
        pltpu.SemaphoreType.DMA,
    ],
)


def _mlp_body(x_ref, agg_ref, w1_ref, w2_ref, b_ref, o_ref):
    acc = jnp.dot(x_ref[...], w1_ref[...], preferred_element_type=jnp.float32)
    acc = acc + jnp.dot(agg_ref[0] + agg_ref[1], w2_ref[...],
                        preferred_element_type=jnp.float32)
    o_ref[...] = jnp.maximum(acc + b_ref[...], 0.0)


def kernel(x, edge_index, edge_weight, W, b):
    ei = edge_index.astype(jnp.int32)
    w = edge_weight.astype(jnp.float32)

    xb = jax.lax.bitcast_convert_type(
        x.astype(jnp.bfloat16)
        .reshape(N, D // 32, 2, 16)
        .transpose(0, 1, 3, 2)
        .reshape(N, D // 2, 2),
        jnp.int32)

    pad = EPAD - E
    fill = (jnp.arange(pad, dtype=jnp.int32) * 97) % N
    ei_p = jnp.concatenate([ei, jnp.stack([fill, fill])], axis=1)
    w_p = jnp.concatenate([w, jnp.zeros((pad,), jnp.float32)])

    agg = _sc_call(xb, ei_p, w_p)

    w1 = W[:D]
    w2 = W[D:]
    b2 = b.reshape(1, D)
    rows_blk = 1000
    out = pl.pallas_call(
        _mlp_body,
        grid=(N // rows_blk,),
        in_specs=[
            pl.BlockSpec((rows_blk, D), lambda i: (i, 0)),
            pl.BlockSpec((NC, rows_blk, D), lambda i: (0, i, 0)),
            pl.BlockSpec((D, D), lambda i: (0, 0)),
            pl.BlockSpec((D, D), lambda i: (0, 0)),
            pl.BlockSpec((1, D), lambda i: (0, 0)),
        ],
        out_specs=pl.BlockSpec((rows_blk, D), lambda i: (i, 0)),
        out_shape=jax.ShapeDtypeStruct((N, D), jnp.float32),
    )(x, agg, w1, w2, b2)
    return out
